# baseline (speedup 1.0000x reference)
import functools

import jax
import jax.numpy as jnp
from jax import lax
from jax.experimental import pallas as pl
from jax.experimental.pallas import tpu as pltpu
from jax.experimental.pallas import tpu_sc as plsc

B = 4096
NROW = 72
D = 256
NC = 2
NS = 16
NW = NC * NS
PER_W = B // NW
CB = 32
NCHUNK = PER_W // CB

_mesh = plsc.VectorSubcoreMesh(core_axis_name="c", subcore_axis_name="s")


@functools.partial(
    pl.kernel,
    out_type=jax.ShapeDtypeStruct((B, 6, D), jnp.float32),
    mesh=_mesh,
    scratch_types=[
        pltpu.SemaphoreType.DMA,
    ],
    compiler_params=pltpu.CompilerParams(use_tc_tiling_on_sc=False),
)
def _select_hands(x_hbm, out_hbm, sem):
    wid = lax.axis_index("s") * NC + lax.axis_index("c")
    base = wid * PER_W
    h1 = pltpu.async_copy(
        x_hbm.at[pl.ds(base, PER_W), pl.ds(27, 3)],
        out_hbm.at[pl.ds(base, PER_W), pl.ds(0, 3)],
        sem,
    )
    h2 = pltpu.async_copy(
        x_hbm.at[pl.ds(base, PER_W), pl.ds(39, 3)],
        out_hbm.at[pl.ds(base, PER_W), pl.ds(3, 3)],
        sem,
    )
    h1.wait()
    h2.wait()


def kernel(input):
    return _select_hands(input)


# speedup vs baseline: 3.4957x; 3.4957x over previous
import functools

import jax
import jax.numpy as jnp
from jax import lax
from jax.experimental import pallas as pl
from jax.experimental.pallas import tpu as pltpu
from jax.experimental.pallas import tpu_sc as plsc

B = 4096
NROW = 72
D = 256
NC = 2
NS = 16
NW = NC * NS
PER_W = B // NW
CB = 32
NCHUNK = PER_W // CB

_mesh = plsc.VectorSubcoreMesh(core_axis_name="c", subcore_axis_name="s")


@functools.partial(
    pl.kernel,
    out_type=jax.ShapeDtypeStruct((B, 6, D), jnp.float32),
    mesh=_mesh,
    scratch_types=[
        pltpu.VMEM((2, CB, 6, D), jnp.float32),
        pltpu.SemaphoreType.DMA,
        pltpu.SemaphoreType.DMA,
        pltpu.SemaphoreType.DMA,
        pltpu.SemaphoreType.DMA,
    ],
    compiler_params=pltpu.CompilerParams(use_tc_tiling_on_sc=False),
)
def _select_hands(x_hbm, out_hbm, buf, sem_in0, sem_in1, sem_out0, sem_out1):
    wid = lax.axis_index("s") * NC + lax.axis_index("c")
    sems_in = (sem_in0, sem_in1)
    sems_out = (sem_out0, sem_out1)

    def start_in(g, slot):
        base = wid * PER_W + g * CB
        h1 = pltpu.async_copy(
            x_hbm.at[pl.ds(base, CB), pl.ds(27, 3)],
            buf.at[slot, :, pl.ds(0, 3)],
            sems_in[slot],
        )
        h2 = pltpu.async_copy(
            x_hbm.at[pl.ds(base, CB), pl.ds(39, 3)],
            buf.at[slot, :, pl.ds(3, 3)],
            sems_in[slot],
        )
        return (h1, h2)

    def start_out(g, slot):
        base = wid * PER_W + g * CB
        return pltpu.async_copy(
            buf.at[slot], out_hbm.at[pl.ds(base, CB)], sems_out[slot]
        )

    in_h = [None, None]
    out_h = [None, None]
    in_h[0] = start_in(0, 0)
    for g in range(NCHUNK):
        slot = g % 2
        nslot = (g + 1) % 2
        if g + 1 < NCHUNK:
            if out_h[nslot] is not None:
                out_h[nslot].wait()
            in_h[nslot] = start_in(g + 1, nslot)
        in_h[slot][0].wait()
        in_h[slot][1].wait()
        out_h[slot] = start_out(g, slot)
    out_h[0].wait()
    out_h[1].wait()


def kernel(input):
    return _select_hands(input)


# speedup vs baseline: 11.4182x; 3.2664x over previous
import functools

import jax
import jax.numpy as jnp
from jax import lax
from jax.experimental import pallas as pl
from jax.experimental.pallas import tpu as pltpu
from jax.experimental.pallas import tpu_sc as plsc

B = 4096
NROW = 72
D = 256
NSEL = 6
NC = 2
NS = 16
NW = NC * NS
IDX_PER_W = B * NSEL // NW
RPC = 128
NCHUNK = IDX_PER_W // RPC

_HANDS = (27, 28, 29, 39, 40, 41)

_mesh = plsc.VectorSubcoreMesh(core_axis_name="c", subcore_axis_name="s")


@functools.partial(
    pl.kernel,
    out_type=jax.ShapeDtypeStruct((B * NSEL, D), jnp.float32),
    mesh=_mesh,
    scratch_types=[
        pltpu.VMEM((IDX_PER_W,), jnp.int32),
        pltpu.VMEM((2, RPC, D), jnp.float32),
        pltpu.SemaphoreType.DMA,
        pltpu.SemaphoreType.DMA,
        pltpu.SemaphoreType.DMA,
        pltpu.SemaphoreType.DMA,
    ],
)
def _select_hands(x_hbm, idx_hbm, out_hbm, idx_v, rowbuf, sg0, sg1, so0, so1):
    wid = lax.axis_index("s") * NC + lax.axis_index("c")
    sems_g = (sg0, sg1)
    sems_o = (so0, so1)

    pltpu.sync_copy(idx_hbm.at[pl.ds(wid * IDX_PER_W, IDX_PER_W)], idx_v)

    def start_gather(g, slot):
        return pltpu.async_copy(
            x_hbm.at[idx_v.at[pl.ds(g * RPC, RPC)]],
            rowbuf.at[slot],
            sems_g[slot],
        )

    def start_out(g, slot):
        return pltpu.async_copy(
            rowbuf.at[slot],
            out_hbm.at[pl.ds(wid * IDX_PER_W + g * RPC, RPC)],
            sems_o[slot],
        )

    gather_h = [None, None]
    out_h = [None, None]
    gather_h[0] = start_gather(0, 0)
    for g in range(NCHUNK):
        slot = g % 2
        nslot = (g + 1) % 2
        gather_h[slot].wait()
        if g + 1 < NCHUNK:
            if out_h[nslot] is not None:
                out_h[nslot].wait()
            gather_h[nslot] = start_gather(g + 1, nslot)
        out_h[slot] = start_out(g, slot)
    out_h[0].wait()
    out_h[1].wait()


_ROW_IDS = (
    jnp.arange(B, dtype=jnp.int32)[:, None] * NROW
    + jnp.array(_HANDS, dtype=jnp.int32)[None, :]
).reshape(-1)


def kernel(input):
    x2d = input.reshape(B * NROW, D)
    return _select_hands(x2d, _ROW_IDS).reshape(B, NSEL, D)


# speedup vs baseline: 14.5365x; 1.2731x over previous
import functools

import jax
import jax.numpy as jnp
from jax import lax
from jax.experimental import pallas as pl
from jax.experimental.pallas import tpu as pltpu
from jax.experimental.pallas import tpu_sc as plsc

B = 4096
NROW = 72
D = 256
NSEL = 6
RPB = 8
NC = 2
NS = 16
NW = NC * NS
PER_W = B // NW
CB = 16
NCHUNK = PER_W // CB
RPC = CB * RPB
IDX_PER_W = PER_W * RPB

_HANDS = (27, 28, 29, 39, 40, 41, 41, 41)

_mesh = plsc.VectorSubcoreMesh(core_axis_name="c", subcore_axis_name="s")


@functools.partial(
    pl.kernel,
    out_type=jax.ShapeDtypeStruct((B, NSEL, D), jnp.float32),
    mesh=_mesh,
    scratch_types=[
        pltpu.VMEM((IDX_PER_W,), jnp.int32),
        pltpu.VMEM((2, RPC, D), jnp.float32),
        pltpu.SemaphoreType.DMA,
        pltpu.SemaphoreType.DMA,
        pltpu.SemaphoreType.DMA,
        pltpu.SemaphoreType.DMA,
    ],
)
def _select_hands(x_hbm, idx_hbm, out_hbm, idx_v, rowbuf, sg0, sg1, so0, so1):
    wid = lax.axis_index("s") * NC + lax.axis_index("c")
    sems_g = (sg0, sg1)
    sems_o = (so0, so1)

    pltpu.sync_copy(idx_hbm.at[pl.ds(wid * IDX_PER_W, IDX_PER_W)], idx_v)

    def start_gather(g, slot):
        return pltpu.async_copy(
            x_hbm.at[idx_v.at[pl.ds(g * RPC, RPC)]],
            rowbuf.at[slot],
            sems_g[slot],
        )

    def start_outs(g, slot):
        base = wid * PER_W + g * CB
        return [
            pltpu.async_copy(
                rowbuf.at[slot, pl.ds(i * RPB, NSEL)],
                out_hbm.at[base + i],
                sems_o[slot],
            )
            for i in range(CB)
        ]

    gather_h = [None, None]
    out_h = [None, None]
    gather_h[0] = start_gather(0, 0)
    for g in range(NCHUNK):
        slot = g % 2
        nslot = (g + 1) % 2
        gather_h[slot].wait()
        if g + 1 < NCHUNK:
            if out_h[nslot] is not None:
                for h in out_h[nslot]:
                    h.wait()
            gather_h[nslot] = start_gather(g + 1, nslot)
        out_h[slot] = start_outs(g, slot)
    for hs in out_h:
        if hs is not None:
            for h in hs:
                h.wait()


_ROW_IDS = (
    jnp.arange(B, dtype=jnp.int32)[:, None] * NROW
    + jnp.array(_HANDS, dtype=jnp.int32)[None, :]
).reshape(-1)


def kernel(input):
    x2d = input.reshape(B * NROW, D)
    return _select_hands(x2d, _ROW_IDS)


# speedup vs baseline: 24.6273x; 1.6942x over previous
import functools

import jax
import jax.numpy as jnp
import numpy as np
from jax import lax
from jax.experimental import pallas as pl
from jax.experimental.pallas import tpu as pltpu
from jax.experimental.pallas import tpu_sc as plsc

B = 4096
NROW = 72
D = 256
NSEL = 6
NC = 2
NS = 16
NW = NC * NS
PER_W = B // NW
IDX_PER_W = NSEL * PER_W

_HANDS = (27, 28, 29, 39, 40, 41)

_mesh = plsc.VectorSubcoreMesh(core_axis_name="c", subcore_axis_name="s")


@functools.partial(
    pl.kernel,
    out_type=jax.ShapeDtypeStruct((NSEL, B, D), jnp.float32),
    mesh=_mesh,
    scratch_types=[
        pltpu.VMEM((IDX_PER_W,), jnp.int32),
        pltpu.VMEM((2, PER_W, D), jnp.float32),
        pltpu.SemaphoreType.DMA,
        pltpu.SemaphoreType.DMA,
        pltpu.SemaphoreType.DMA,
        pltpu.SemaphoreType.DMA,
    ],
)
def _select_hands(x_hbm, idx_hbm, out_hbm, idx_v, rowbuf, sg0, sg1, so0, so1):
    wid = lax.axis_index("s") * NC + lax.axis_index("c")
    wb = wid * PER_W
    sems_g = (sg0, sg1)
    sems_o = (so0, so1)

    for j in range(NSEL):
        pltpu.sync_copy(
            idx_hbm.at[pl.ds(j * B + wb, PER_W)],
            idx_v.at[pl.ds(j * PER_W, PER_W)],
        )

    def start_gather(j, slot):
        return pltpu.async_copy(
            x_hbm.at[idx_v.at[pl.ds(j * PER_W, PER_W)]],
            rowbuf.at[slot],
            sems_g[slot],
        )

    def start_out(j, slot):
        return pltpu.async_copy(
            rowbuf.at[slot],
            out_hbm.at[j, pl.ds(wb, PER_W)],
            sems_o[slot],
        )

    gather_h = [None, None]
    out_h = [None, None]
    gather_h[0] = start_gather(0, 0)
    for j in range(NSEL):
        slot = j % 2
        nslot = (j + 1) % 2
        gather_h[slot].wait()
        if j + 1 < NSEL:
            if out_h[nslot] is not None:
                out_h[nslot].wait()
            gather_h[nslot] = start_gather(j + 1, nslot)
        out_h[slot] = start_out(j, slot)
    out_h[0].wait()
    out_h[1].wait()


_ROW_IDS = (
    np.array(_HANDS, dtype=np.int32)[:, None]
    + np.arange(B, dtype=np.int32)[None, :] * NROW
).reshape(-1)


def kernel(input):
    x2d = input.reshape(B * NROW, D)
    planes = _select_hands(x2d, _ROW_IDS)
    return jnp.swapaxes(planes, 0, 1)


# speedup vs baseline: 27.3580x; 1.1109x over previous
import functools

import jax
import jax.numpy as jnp
import numpy as np
from jax import lax
from jax.experimental import pallas as pl
from jax.experimental.pallas import tpu as pltpu
from jax.experimental.pallas import tpu_sc as plsc

B = 4096
NROW = 72
D = 256
NSEL = 6
NC = 2
NS = 16
NW = NC * NS
PER_W = B // NW
IDX_PER_W = NSEL * PER_W
NSLOT = 3

_HANDS = (27, 28, 29, 39, 40, 41)

_mesh = plsc.VectorSubcoreMesh(core_axis_name="c", subcore_axis_name="s")


@functools.partial(
    pl.kernel,
    out_type=jax.ShapeDtypeStruct((NSEL, B, D), jnp.float32),
    mesh=_mesh,
    scratch_types=[
        pltpu.VMEM((IDX_PER_W,), jnp.int32),
        pltpu.VMEM((NSLOT, PER_W, D), jnp.float32),
        pltpu.SemaphoreType.DMA,
        pltpu.SemaphoreType.DMA,
        pltpu.SemaphoreType.DMA,
        pltpu.SemaphoreType.DMA,
        pltpu.SemaphoreType.DMA,
        pltpu.SemaphoreType.DMA,
    ],
)
def _select_hands(x_hbm, idx_hbm, out_hbm, idx_v, rowbuf,
                  sg0, sg1, sg2, so0, so1, so2):
    wid = lax.axis_index("s") * NC + lax.axis_index("c")
    wb = wid * PER_W
    sems_g = (sg0, sg1, sg2)
    sems_o = (so0, so1, so2)

    pltpu.sync_copy(idx_hbm.at[pl.ds(wid * IDX_PER_W, IDX_PER_W)], idx_v)

    def start_gather(j, slot):
        return pltpu.async_copy(
            x_hbm.at[idx_v.at[pl.ds(j * PER_W, PER_W)]],
            rowbuf.at[slot],
            sems_g[slot],
        )

    def start_out(j, slot):
        return pltpu.async_copy(
            rowbuf.at[slot],
            out_hbm.at[j, pl.ds(wb, PER_W)],
            sems_o[slot],
        )

    gather_h = [None] * NSLOT
    out_h = [None] * NSLOT
    gather_h[0] = start_gather(0, 0)
    gather_h[1] = start_gather(1, 1)
    for j in range(NSEL):
        slot = j % NSLOT
        nslot = (j + 2) % NSLOT
        if j + 2 < NSEL:
            if out_h[nslot] is not None:
                out_h[nslot].wait()
            gather_h[nslot] = start_gather(j + 2, nslot)
        gather_h[slot].wait()
        out_h[slot] = start_out(j, slot)
    for h in out_h:
        h.wait()


_ROW_IDS = (
    np.arange(NW, dtype=np.int32)[:, None, None] * 0
    + np.array(_HANDS, dtype=np.int32)[None, :, None]
    + (
        np.arange(NW, dtype=np.int32)[:, None, None] * PER_W
        + np.arange(PER_W, dtype=np.int32)[None, None, :]
    ) * NROW
).reshape(-1)


def kernel(input):
    x2d = input.reshape(B * NROW, D)
    planes = _select_hands(x2d, _ROW_IDS)
    return jnp.swapaxes(planes, 0, 1)


# speedup vs baseline: 27.7644x; 1.0149x over previous
import functools

import jax
import jax.numpy as jnp
import numpy as np
from jax import lax
from jax.experimental import pallas as pl
from jax.experimental.pallas import tpu as pltpu
from jax.experimental.pallas import tpu_sc as plsc

B = 4096
NROW = 72
D = 256
NSEL = 6
NC = 2
NS = 16
NW = NC * NS
PER_W = B // NW
IDX_PER_W = NSEL * PER_W
NSLOT = 3

_HANDS = (27, 28, 29, 39, 40, 41)

_mesh = plsc.VectorSubcoreMesh(core_axis_name="c", subcore_axis_name="s")


@functools.partial(
    pl.kernel,
    out_type=jax.ShapeDtypeStruct((NSEL, B, D), jnp.float32),
    mesh=_mesh,
    scratch_types=[
        pltpu.VMEM((IDX_PER_W,), jnp.int32),
        pltpu.VMEM((NSLOT, PER_W, D), jnp.float32),
        pltpu.SemaphoreType.DMA,
        pltpu.SemaphoreType.DMA,
        pltpu.SemaphoreType.DMA,
        pltpu.SemaphoreType.DMA,
        pltpu.SemaphoreType.DMA,
        pltpu.SemaphoreType.DMA,
    ],
)
def _select_hands(x_hbm, out_hbm, idx_v, rowbuf,
                  sg0, sg1, sg2, so0, so1, so2):
    wid = lax.axis_index("s") * NC + lax.axis_index("c")
    wb = wid * PER_W
    sems_g = (sg0, sg1, sg2)
    sems_o = (so0, so1, so2)

    ramp = lax.iota(jnp.int32, 16)
    for j in range(NSEL):
        for k in range(PER_W // 16):
            idx_v[pl.ds((j * (PER_W // 16) + k) * 16, 16)] = (
                (wb + k * 16 + ramp) * NROW + _HANDS[j]
            )

    def start_gather(j, slot):
        return pltpu.async_copy(
            x_hbm.at[idx_v.at[pl.ds(j * PER_W, PER_W)]],
            rowbuf.at[slot],
            sems_g[slot],
        )

    def start_out(j, slot):
        return pltpu.async_copy(
            rowbuf.at[slot],
            out_hbm.at[j, pl.ds(wb, PER_W)],
            sems_o[slot],
        )

    gather_h = [None] * NSLOT
    out_h = [None] * NSLOT
    gather_h[0] = start_gather(0, 0)
    gather_h[1] = start_gather(1, 1)
    for j in range(NSEL):
        slot = j % NSLOT
        nslot = (j + 2) % NSLOT
        if j + 2 < NSEL:
            if out_h[nslot] is not None:
                out_h[nslot].wait()
            gather_h[nslot] = start_gather(j + 2, nslot)
        gather_h[slot].wait()
        out_h[slot] = start_out(j, slot)
    for h in out_h:
        h.wait()


def kernel(input):
    x2d = input.reshape(B * NROW, D)
    planes = _select_hands(x2d)
    return jnp.swapaxes(planes, 0, 1)
